# baseline (speedup 1.0000x reference)
import functools

import jax
import jax.numpy as jnp
from jax import lax
from jax.experimental import pallas as pl
from jax.experimental.pallas import tpu as pltpu
from jax.experimental.pallas import tpu_sc as plsc

B = 4096
S = 200
E = 64
L = 16
NC = 2
NS = 16
NW = NC * NS
BW = B // NW
NBUF = 4
SCALE = 8.0


def _sc_embed(src5, token_table, pos_t):
    mesh = plsc.VectorSubcoreMesh(core_axis_name="c", subcore_axis_name="s")

    @functools.partial(
        pl.kernel,
        mesh=mesh,
        compiler_params=pltpu.CompilerParams(
            use_tc_tiling_on_sc=False, needs_layout_passes=False),
        out_type=jax.ShapeDtypeStruct((S, E // 8, B // 128, 8, 128),
                                      jnp.float32),
        scratch_types=[
            pltpu.VMEM((S // 8, 8, BW), jnp.int32),
            pltpu.VMEM((S, E), jnp.float32),
            pltpu.VMEM((NBUF, BW, E), jnp.float32),
            pltpu.VMEM((2, E // 8, 8, 128), jnp.float32),
            [pltpu.SemaphoreType.DMA] * NBUF,
            [pltpu.SemaphoreType.DMA] * 2,
        ],
    )
    def body(src_hbm, tab_hbm, pos_hbm, out_hbm, idx_v, pos_v, ring, obuf,
             gsems, osems):
        w = lax.axis_index("s") * NC + lax.axis_index("c")
        pltpu.sync_copy(src_hbm.at[:, w], idx_v)
        pltpu.sync_copy(pos_hbm, pos_v)

        iota = jax.lax.iota(jnp.int32, L)
        rows = [iota + j * L for j in range(8)]

        def start_gather(s, slot):
            pltpu.async_copy(
                tab_hbm.at[idx_v.at[s // 8, s % 8]], ring.at[slot],
                gsems[slot])

        def wait_gather(slot):
            pltpu.make_async_copy(
                tab_hbm.at[pl.ds(0, BW)], ring.at[slot], gsems[slot]).wait()

        def wait_write(ob):
            pltpu.make_async_copy(
                obuf.at[ob], out_hbm.at[0, :, w], osems[ob]).wait()

        start_gather(0, 0)
        start_gather(1, 1)

        @pl.loop(0, S, step=NBUF)
        def _positions(j):
            for k in range(NBUF):
                s = j + k
                ob = k % 2
                wait_gather(k)

                @pl.when(s >= 2)
                def _():
                    wait_write(ob)

                @pl.loop(0, E)
                def _feat(e, _k=k, _ob=ob, _s=s):
                    cols = jnp.full((L,), e, jnp.int32)
                    pvec = plsc.load_gather(
                        pos_v, [jnp.full((L,), _s, jnp.int32), cols])
                    for jj in range(8):
                        val = plsc.load_gather(ring.at[_k], [rows[jj], cols])
                        obuf[_ob, e // 8, e % 8, pl.ds(jj * L, L)] = (
                            val * SCALE + pvec)

                pltpu.async_copy(obuf.at[ob], out_hbm.at[s, :, w], osems[ob])

                @pl.when(s + 2 < S)
                def _():
                    start_gather(s + 2, (k + 2) % NBUF)

        wait_write(0)
        wait_write(1)

    return body(src5, token_table, pos_t)


def kernel(src, tgt, token_table, pos_table):
    del tgt
    src5 = src.T.reshape(S // 8, 8, B // 128, 128).transpose(0, 2, 1, 3)
    out5 = _sc_embed(src5, token_table, pos_table)
    return out5.transpose(2, 4, 0, 1, 3).reshape(B, S, E)


# speedup vs baseline: 1.0675x; 1.0675x over previous
import functools

import jax
import jax.numpy as jnp
from jax import lax
from jax.experimental import pallas as pl
from jax.experimental.pallas import tpu as pltpu
from jax.experimental.pallas import tpu_sc as plsc

B = 4096
S = 200
E = 64
V = 1000000
L = 16
NC = 2
NS = 16
NW = NC * NS
BW = B // NW
NBUF = 4
SCALE = 8.0


def _sc_embed(src5, table2, pos_table):
    mesh = plsc.VectorSubcoreMesh(core_axis_name="c", subcore_axis_name="s")

    @functools.partial(
        pl.kernel,
        mesh=mesh,
        compiler_params=pltpu.CompilerParams(
            use_tc_tiling_on_sc=False, needs_layout_passes=False),
        out_type=jax.ShapeDtypeStruct((B // 8, S // 2, 8, 2, E),
                                      jnp.float32),
        scratch_types=[
            pltpu.VMEM((S // 8, 8, BW), jnp.int32),
            pltpu.VMEM((S, E), jnp.float32),
            pltpu.VMEM((NBUF, BW, E), jnp.float32),
            pltpu.VMEM((2, BW // 8, 8, E), jnp.float32),
            [pltpu.SemaphoreType.DMA] * NBUF,
            [pltpu.SemaphoreType.DMA] * 2,
        ],
    )
    def body(src_hbm, tab_hbm, pos_hbm, out_hbm, idx_v, pos_v, ring, obuf,
             gsems, osems):
        w = lax.axis_index("s") * NC + lax.axis_index("c")
        pltpu.sync_copy(src_hbm.at[:, w], idx_v)
        pltpu.sync_copy(pos_hbm, pos_v)

        @pl.loop(0, S // 8)
        def _dbl(ts):
            for ss in range(8):
                for jj in range(BW // L):
                    sl = (ts, ss, pl.ds(jj * L, L))
                    idx_v[sl] = idx_v[sl] * 2

        def start_gather(s, slot):
            pltpu.async_copy(
                tab_hbm.at[idx_v.at[s // 8, s % 8]], ring.at[slot],
                gsems[slot])

        def wait_gather(slot):
            pltpu.make_async_copy(
                tab_hbm.at[pl.ds(0, BW)], ring.at[slot], gsems[slot]).wait()

        def out_window(s):
            return out_hbm.at[pl.ds(w * (BW // 8), BW // 8), s // 2, :,
                              s % 2, :]

        def wait_write(ob):
            pltpu.make_async_copy(
                obuf.at[ob], out_window(0), osems[ob]).wait()

        start_gather(0, 0)
        start_gather(1, 1)

        @pl.loop(0, S, step=NBUF)
        def _positions(j):
            for k in range(NBUF):
                s = j + k
                ob = k % 2
                wait_gather(k)

                @pl.when(s >= 2)
                def _():
                    wait_write(ob)

                pv = [pos_v[s, pl.ds(c * L, L)] for c in range(E // L)]

                @pl.loop(0, BW, step=8)
                def _rows(r, _k=k, _ob=ob, _pv=pv):
                    rb = r // 8
                    for dr in range(8):
                        for c in range(E // L):
                            obuf[_ob, rb, dr, pl.ds(c * L, L)] = (
                                ring[_k, r + dr, pl.ds(c * L, L)] * SCALE
                                + _pv[c])

                pltpu.async_copy(obuf.at[ob], out_window(s), osems[ob])

                @pl.when(s + 2 < S)
                def _():
                    start_gather(s + 2, (k + 2) % NBUF)

        wait_write(0)
        wait_write(1)

    return body(src5, table2, pos_table)


def kernel(src, tgt, token_table, pos_table):
    del tgt
    src5 = src.T.reshape(S // 8, 8, B // 128, 128).transpose(0, 2, 1, 3)
    table2 = jnp.pad(token_table, ((0, 0), (0, E))).reshape(2 * V, E)
    out6 = _sc_embed(src5, table2, pos_table)
    out = out6.transpose(0, 2, 1, 3, 4).reshape(B, S, E)
    return out


# speedup vs baseline: 1.9591x; 1.8351x over previous
import functools

import jax
import jax.numpy as jnp
from jax import lax
from jax.experimental import pallas as pl
from jax.experimental.pallas import tpu as pltpu
from jax.experimental.pallas import tpu_sc as plsc

B = 4096
S = 200
E = 64
V = 1000000
L = 16
NC = 2
NS = 16
NW = NC * NS
BW = B // NW
NBUF = 4
SCALE = 8.0


def _sc_embed(src5, table2, pos_table):
    mesh = plsc.VectorSubcoreMesh(core_axis_name="c", subcore_axis_name="s")

    @functools.partial(
        pl.kernel,
        mesh=mesh,
        compiler_params=pltpu.CompilerParams(
            use_tc_tiling_on_sc=False, needs_layout_passes=False),
        out_type=jax.ShapeDtypeStruct((B, S * E), jnp.float32),
        scratch_types=[
            pltpu.VMEM((S // 8, 8, BW), jnp.int32),
            pltpu.VMEM((S, E), jnp.float32),
            pltpu.VMEM((NBUF, BW, E), jnp.float32),
            [pltpu.SemaphoreType.DMA] * NBUF,
            [pltpu.SemaphoreType.DMA] * NBUF,
        ],
    )
    def body(src_hbm, tab_hbm, pos_hbm, out_hbm, idx_v, pos_v, ring,
             gsems, osems):
        w = lax.axis_index("s") * NC + lax.axis_index("c")
        pltpu.sync_copy(src_hbm.at[:, w], idx_v)
        pltpu.sync_copy(pos_hbm, pos_v)

        @pl.loop(0, S // 8)
        def _dbl(ts):
            for ss in range(8):
                for jj in range(BW // L):
                    sl = (ts, ss, pl.ds(jj * L, L))
                    idx_v[sl] = idx_v[sl] * 2

        def start_gather(s, slot):
            pltpu.async_copy(
                tab_hbm.at[idx_v.at[s // 8, s % 8]], ring.at[slot],
                gsems[slot])

        def wait_gather(slot):
            pltpu.make_async_copy(
                tab_hbm.at[pl.ds(0, BW)], ring.at[slot], gsems[slot]).wait()

        def out_window(s):
            return out_hbm.at[pl.ds(w * BW, BW), pl.ds(s * E, E)]

        def wait_write(slot):
            pltpu.make_async_copy(
                ring.at[slot], out_window(0), osems[slot]).wait()

        start_gather(0, 0)
        start_gather(1, 1)

        @pl.loop(0, S, step=NBUF)
        def _positions(j):
            for k in range(NBUF):
                s = j + k
                wait_gather(k)

                pv = [pos_v[s, pl.ds(c * L, L)] for c in range(E // L)]

                @pl.loop(0, BW, step=4)
                def _rows(r, _k=k, _pv=pv):
                    for dr in range(4):
                        for c in range(E // L):
                            sl = (_k, r + dr, pl.ds(c * L, L))
                            ring[sl] = ring[sl] * SCALE + _pv[c]

                pltpu.async_copy(ring.at[k], out_window(s), osems[k])

                nxt = s + 2
                nslot = (k + 2) % NBUF

                @pl.when(nxt < S)
                def _():
                    @pl.when(s >= 2)
                    def _():
                        wait_write(nslot)

                    start_gather(nxt, nslot)

        for k in range(NBUF):
            wait_write(k)

    return body(src5, table2, pos_table)


def kernel(src, tgt, token_table, pos_table):
    del tgt
    src5 = src.T.reshape(S // 8, 8, B // 128, 128).transpose(0, 2, 1, 3)
    table2 = jnp.pad(token_table, ((0, 0), (0, E))).reshape(2 * V, E)
    out2 = _sc_embed(src5, table2, pos_table)
    return out2.reshape(B, S, E)


# speedup vs baseline: 2.0554x; 1.0492x over previous
import functools

import jax
import jax.numpy as jnp
from jax import lax
from jax.experimental import pallas as pl
from jax.experimental.pallas import tpu as pltpu
from jax.experimental.pallas import tpu_sc as plsc

B = 4096
S = 200
E = 64
V = 1000000
L = 16
NC = 2
NS = 16
NW = NC * NS
BW = B // NW
NBUF = 4
SCALE = 8.0


def _format_table(tab_t):
    CB = 1920

    def body(t_ref, o_ref):
        t = t_ref[...].T
        o_ref[...] = jnp.concatenate(
            [t, jnp.zeros((CB, 128 - E), jnp.float32)], axis=1)

    return pl.pallas_call(
        body,
        grid=(pl.cdiv(V, CB),),
        in_specs=[pl.BlockSpec((E, CB), lambda i: (0, i))],
        out_specs=pl.BlockSpec((CB, 128), lambda i: (i, 0)),
        out_shape=jax.ShapeDtypeStruct((V, 128), jnp.float32),
    )(tab_t)


def _sc_embed(src5, table2, pos_table):
    mesh = plsc.VectorSubcoreMesh(core_axis_name="c", subcore_axis_name="s")

    @functools.partial(
        pl.kernel,
        mesh=mesh,
        compiler_params=pltpu.CompilerParams(
            use_tc_tiling_on_sc=False, needs_layout_passes=False),
        out_type=jax.ShapeDtypeStruct((B, S * E), jnp.float32),
        scratch_types=[
            pltpu.VMEM((S // 8, 8, BW), jnp.int32),
            pltpu.VMEM((S, E), jnp.float32),
            pltpu.VMEM((NBUF, BW, E), jnp.float32),
            [pltpu.SemaphoreType.DMA] * NBUF,
            [pltpu.SemaphoreType.DMA] * NBUF,
        ],
    )
    def body(src_hbm, tab_hbm, pos_hbm, out_hbm, idx_v, pos_v, ring,
             gsems, osems):
        w = lax.axis_index("s") * NC + lax.axis_index("c")
        pltpu.sync_copy(src_hbm.at[:, w], idx_v)
        pltpu.sync_copy(pos_hbm, pos_v)

        @pl.loop(0, S // 8)
        def _dbl(ts):
            for ss in range(8):
                for jj in range(BW // L):
                    sl = (ts, ss, pl.ds(jj * L, L))
                    idx_v[sl] = idx_v[sl] * 2

        def start_gather(s, slot):
            pltpu.async_copy(
                tab_hbm.at[idx_v.at[s // 8, s % 8]], ring.at[slot],
                gsems[slot])

        def wait_gather(slot):
            pltpu.make_async_copy(
                tab_hbm.at[pl.ds(0, BW)], ring.at[slot], gsems[slot]).wait()

        def out_window(s):
            return out_hbm.at[pl.ds(w * BW, BW), pl.ds(s * E, E)]

        def wait_write(slot):
            pltpu.make_async_copy(
                ring.at[slot], out_window(0), osems[slot]).wait()

        start_gather(0, 0)
        start_gather(1, 1)

        @pl.loop(0, S, step=NBUF)
        def _positions(j):
            for k in range(NBUF):
                s = j + k
                wait_gather(k)

                pv = [pos_v[s, pl.ds(c * L, L)] for c in range(E // L)]

                @pl.loop(0, BW, step=4)
                def _rows(r, _k=k, _pv=pv):
                    for dr in range(4):
                        for c in range(E // L):
                            sl = (_k, r + dr, pl.ds(c * L, L))
                            ring[sl] = ring[sl] * SCALE + _pv[c]

                pltpu.async_copy(ring.at[k], out_window(s), osems[k])

                nxt = s + 2
                nslot = (k + 2) % NBUF

                @pl.when(nxt < S)
                def _():
                    @pl.when(s >= 2)
                    def _():
                        wait_write(nslot)

                    start_gather(nxt, nslot)

        for k in range(NBUF):
            wait_write(k)

    return body(src5, table2, pos_table)


def kernel(src, tgt, token_table, pos_table):
    del tgt
    src5 = src.T.reshape(S // 8, 8, B // 128, 128).transpose(0, 2, 1, 3)
    table2 = _format_table(token_table.T).reshape(2 * V, E)
    out2 = _sc_embed(src5, table2, pos_table)
    return out2.reshape(B, S, E)


# speedup vs baseline: 2.6121x; 1.2709x over previous
import functools

import jax
import jax.numpy as jnp
from jax import lax
from jax.experimental import pallas as pl
from jax.experimental.pallas import tpu as pltpu
from jax.experimental.pallas import tpu_sc as plsc

B = 4096
S = 200
E = 64
V = 1000000
L = 16
NC = 2
NS = 16
NW = NC * NS
BW = B // NW
NBUF = 4
SCALE = 8.0


def _format_table(tab_t):
    CB = 7680

    def body(t_ref, o_ref):
        o_ref[:, 0:E] = t_ref[...].T
        o_ref[:, E:128] = jnp.zeros((CB, 128 - E), jnp.float32)

    return pl.pallas_call(
        body,
        grid=(pl.cdiv(V, CB),),
        in_specs=[pl.BlockSpec((E, CB), lambda i: (0, i))],
        out_specs=pl.BlockSpec((CB, 128), lambda i: (i, 0)),
        out_shape=jax.ShapeDtypeStruct((V, 128), jnp.float32),
    )(tab_t)


def _sc_embed(src5, table2, pos_table):
    mesh = plsc.VectorSubcoreMesh(core_axis_name="c", subcore_axis_name="s")

    @functools.partial(
        pl.kernel,
        mesh=mesh,
        compiler_params=pltpu.CompilerParams(
            use_tc_tiling_on_sc=False, needs_layout_passes=False),
        out_type=jax.ShapeDtypeStruct((B, S * E), jnp.float32),
        scratch_types=[
            pltpu.VMEM((S // 8, 8, BW), jnp.int32),
            pltpu.VMEM((S, E), jnp.float32),
            pltpu.VMEM((NBUF, BW, E), jnp.float32),
            [pltpu.SemaphoreType.DMA] * NBUF,
            [pltpu.SemaphoreType.DMA] * NBUF,
        ],
    )
    def body(src_hbm, tab_hbm, pos_hbm, out_hbm, idx_v, pos_v, ring,
             gsems, osems):
        w = lax.axis_index("s") * NC + lax.axis_index("c")
        pltpu.sync_copy(src_hbm.at[:, w], idx_v)
        pltpu.sync_copy(pos_hbm, pos_v)

        @pl.loop(0, S // 8)
        def _dbl(ts):
            for ss in range(8):
                for jj in range(BW // L):
                    sl = (ts, ss, pl.ds(jj * L, L))
                    idx_v[sl] = idx_v[sl] * 2

        def start_gather(s, slot):
            pltpu.async_copy(
                tab_hbm.at[idx_v.at[s // 8, s % 8]], ring.at[slot],
                gsems[slot])

        def wait_gather(slot):
            pltpu.make_async_copy(
                tab_hbm.at[pl.ds(0, BW)], ring.at[slot], gsems[slot]).wait()

        def out_window(s):
            return out_hbm.at[pl.ds(w * BW, BW), pl.ds(s * E, E)]

        def wait_write(slot):
            pltpu.make_async_copy(
                ring.at[slot], out_window(0), osems[slot]).wait()

        start_gather(0, 0)
        start_gather(1, 1)

        @pl.loop(0, S, step=NBUF)
        def _positions(j):
            for k in range(NBUF):
                s = j + k
                wait_gather(k)

                pv = [pos_v[s, pl.ds(c * L, L)] for c in range(E // L)]

                @pl.loop(0, BW, step=4)
                def _rows(r, _k=k, _pv=pv):
                    for dr in range(4):
                        for c in range(E // L):
                            sl = (_k, r + dr, pl.ds(c * L, L))
                            ring[sl] = ring[sl] * SCALE + _pv[c]

                pltpu.async_copy(ring.at[k], out_window(s), osems[k])

                nxt = s + 2
                nslot = (k + 2) % NBUF

                @pl.when(nxt < S)
                def _():
                    @pl.when(s >= 2)
                    def _():
                        wait_write(nslot)

                    start_gather(nxt, nslot)

        for k in range(NBUF):
            wait_write(k)

    return body(src5, table2, pos_table)


def kernel(src, tgt, token_table, pos_table):
    del tgt
    src5 = src.T.reshape(S // 8, 8, B // 128, 128).transpose(0, 2, 1, 3)
    table2 = _format_table(token_table.T).reshape(2 * V, E)
    out2 = _sc_embed(src5, table2, pos_table)
    return out2.reshape(B, S, E)


# speedup vs baseline: 2.6800x; 1.0260x over previous
import functools

import jax
import jax.numpy as jnp
from jax import lax
from jax.experimental import pallas as pl
from jax.experimental.pallas import tpu as pltpu
from jax.experimental.pallas import tpu_sc as plsc

B = 4096
S = 200
E = 64
V = 1000000
L = 16
NC = 2
NS = 16
NW = NC * NS
BW = B // NW
NBUF = 4
SCALE = 8.0


def _format_table(tab_t):
    CB = 15360

    def body(t_ref, o_ref):
        o_ref[:, 0:E] = t_ref[...].T
        o_ref[:, E:128] = jnp.zeros((CB, 128 - E), jnp.float32)

    return pl.pallas_call(
        body,
        grid=(pl.cdiv(V, CB),),
        in_specs=[pl.BlockSpec((E, CB), lambda i: (0, i))],
        out_specs=pl.BlockSpec((CB, 128), lambda i: (i, 0)),
        out_shape=jax.ShapeDtypeStruct((V, 128), jnp.float32),
    )(tab_t)


def _sc_embed(src5, table2, pos_table):
    mesh = plsc.VectorSubcoreMesh(core_axis_name="c", subcore_axis_name="s")

    @functools.partial(
        pl.kernel,
        mesh=mesh,
        compiler_params=pltpu.CompilerParams(
            use_tc_tiling_on_sc=False, needs_layout_passes=False),
        out_type=jax.ShapeDtypeStruct((B, S * E), jnp.float32),
        scratch_types=[
            pltpu.VMEM((S // 8, 8, BW), jnp.int32),
            pltpu.VMEM((S, E), jnp.float32),
            pltpu.VMEM((NBUF, BW, E), jnp.float32),
            [pltpu.SemaphoreType.DMA] * NBUF,
            [pltpu.SemaphoreType.DMA] * NBUF,
        ],
    )
    def body(src_hbm, tab_hbm, pos_hbm, out_hbm, idx_v, pos_v, ring,
             gsems, osems):
        w = lax.axis_index("s") * NC + lax.axis_index("c")
        pltpu.sync_copy(src_hbm.at[:, w], idx_v)
        pltpu.sync_copy(pos_hbm, pos_v)

        @pl.loop(0, S // 8)
        def _dbl(ts):
            for ss in range(8):
                for jj in range(BW // L):
                    sl = (ts, ss, pl.ds(jj * L, L))
                    idx_v[sl] = idx_v[sl] * 2

        def start_gather(s, slot):
            pltpu.async_copy(
                tab_hbm.at[idx_v.at[s // 8, s % 8]], ring.at[slot],
                gsems[slot])

        def wait_gather(slot):
            pltpu.make_async_copy(
                tab_hbm.at[pl.ds(0, BW)], ring.at[slot], gsems[slot]).wait()

        def out_window(s):
            return out_hbm.at[pl.ds(w * BW, BW), pl.ds(s * E, E)]

        def wait_write(slot):
            pltpu.make_async_copy(
                ring.at[slot], out_window(0), osems[slot]).wait()

        start_gather(0, 0)
        start_gather(1, 1)

        @pl.loop(0, S, step=NBUF)
        def _positions(j):
            for k in range(NBUF):
                s = j + k
                wait_gather(k)

                pv = [pos_v[s, pl.ds(c * L, L)] for c in range(E // L)]

                @pl.loop(0, BW, step=4)
                def _rows(r, _k=k, _pv=pv):
                    for dr in range(4):
                        for c in range(E // L):
                            sl = (_k, r + dr, pl.ds(c * L, L))
                            ring[sl] = ring[sl] * SCALE + _pv[c]

                pltpu.async_copy(ring.at[k], out_window(s), osems[k])

                nxt = s + 2
                nslot = (k + 2) % NBUF

                @pl.when(nxt < S)
                def _():
                    @pl.when(s >= 2)
                    def _():
                        wait_write(nslot)

                    start_gather(nxt, nslot)

        for k in range(NBUF):
            wait_write(k)

    return body(src5, table2, pos_table)


def kernel(src, tgt, token_table, pos_table):
    del tgt
    src5 = src.T.reshape(S // 8, 8, B // 128, 128).transpose(0, 2, 1, 3)
    table2 = _format_table(token_table.T).reshape(2 * V, E)
    out2 = _sc_embed(src5, table2, pos_table)
    return out2.reshape(B, S, E)


# speedup vs baseline: 2.6986x; 1.0069x over previous
import functools

import jax
import jax.numpy as jnp
from jax import lax
from jax.experimental import pallas as pl
from jax.experimental.pallas import tpu as pltpu
from jax.experimental.pallas import tpu_sc as plsc

B = 4096
S = 200
E = 64
V = 1000000
L = 16
NC = 2
NS = 16
NW = NC * NS
BW = B // NW
NBUF = 4
SCALE = 8.0


def _format_table(tab_t):
    CB = 23040

    def body(t_ref, o_ref):
        o_ref[:, 0:E] = t_ref[...].T
        o_ref[:, E:128] = jnp.zeros((CB, 128 - E), jnp.float32)

    return pl.pallas_call(
        body,
        grid=(pl.cdiv(V, CB),),
        in_specs=[pl.BlockSpec((E, CB), lambda i: (0, i))],
        out_specs=pl.BlockSpec((CB, 128), lambda i: (i, 0)),
        out_shape=jax.ShapeDtypeStruct((V, 128), jnp.float32),
    )(tab_t)


def _sc_embed(src5, table2, pos_table):
    mesh = plsc.VectorSubcoreMesh(core_axis_name="c", subcore_axis_name="s")

    @functools.partial(
        pl.kernel,
        mesh=mesh,
        compiler_params=pltpu.CompilerParams(
            use_tc_tiling_on_sc=False, needs_layout_passes=False),
        out_type=jax.ShapeDtypeStruct((B, S * E), jnp.float32),
        scratch_types=[
            pltpu.VMEM((S // 8, 8, BW), jnp.int32),
            pltpu.VMEM((S, E), jnp.float32),
            pltpu.VMEM((NBUF, BW, E), jnp.float32),
            [pltpu.SemaphoreType.DMA] * NBUF,
            [pltpu.SemaphoreType.DMA] * NBUF,
        ],
    )
    def body(src_hbm, tab_hbm, pos_hbm, out_hbm, idx_v, pos_v, ring,
             gsems, osems):
        w = lax.axis_index("s") * NC + lax.axis_index("c")
        pltpu.sync_copy(src_hbm.at[:, w], idx_v)
        pltpu.sync_copy(pos_hbm, pos_v)

        @pl.loop(0, S // 8)
        def _dbl(ts):
            for ss in range(8):
                for jj in range(BW // L):
                    sl = (ts, ss, pl.ds(jj * L, L))
                    idx_v[sl] = idx_v[sl] * 2

        def start_gather(s, slot):
            pltpu.async_copy(
                tab_hbm.at[idx_v.at[s // 8, s % 8]], ring.at[slot],
                gsems[slot])

        def wait_gather(slot):
            pltpu.make_async_copy(
                tab_hbm.at[pl.ds(0, BW)], ring.at[slot], gsems[slot]).wait()

        def out_window(s):
            return out_hbm.at[pl.ds(w * BW, BW), pl.ds(s * E, E)]

        def wait_write(slot):
            pltpu.make_async_copy(
                ring.at[slot], out_window(0), osems[slot]).wait()

        start_gather(0, 0)
        start_gather(1, 1)

        @pl.loop(0, S, step=NBUF)
        def _positions(j):
            for k in range(NBUF):
                s = j + k
                wait_gather(k)

                pv = [pos_v[s, pl.ds(c * L, L)] for c in range(E // L)]

                @pl.loop(0, BW, step=4)
                def _rows(r, _k=k, _pv=pv):
                    for dr in range(4):
                        for c in range(E // L):
                            sl = (_k, r + dr, pl.ds(c * L, L))
                            ring[sl] = ring[sl] * SCALE + _pv[c]

                pltpu.async_copy(ring.at[k], out_window(s), osems[k])

                nxt = s + 2
                nslot = (k + 2) % NBUF

                @pl.when(nxt < S)
                def _():
                    @pl.when(s >= 2)
                    def _():
                        wait_write(nslot)

                    start_gather(nxt, nslot)

        for k in range(NBUF):
            wait_write(k)

    return body(src5, table2, pos_table)


def kernel(src, tgt, token_table, pos_table):
    del tgt
    src5 = src.T.reshape(S // 8, 8, B // 128, 128).transpose(0, 2, 1, 3)
    table2 = _format_table(token_table.T).reshape(2 * V, E)
    out2 = _sc_embed(src5, table2, pos_table)
    return out2.reshape(B, S, E)
